# baseline (speedup 1.0000x reference)
import functools

import jax
import jax.numpy as jnp
from jax import lax
from jax.experimental import pallas as pl
from jax.experimental.pallas import tpu as pltpu
from jax.experimental.pallas import tpu_sc as plsc

DIM = 32
BATCH = 16384
NC = 2
NS = 16
NW = NC * NS
B_PER_W = BATCH // NW
GATHER_CHUNK = 128
L = 16


def _sc_kernel(user_idx_hbm, item_idx_hbm, user_table_hbm, item_table_hbm,
               out_hbm, uidx_v, iidx_v, u_rows, i_rows, out_v, sem):
    wid = lax.axis_index("s") * NC + lax.axis_index("c")
    base = wid * B_PER_W

    pltpu.sync_copy(user_idx_hbm.at[pl.ds(base, B_PER_W)], uidx_v)
    pltpu.sync_copy(item_idx_hbm.at[pl.ds(base, B_PER_W)], iidx_v)

    copies = []
    for c in range(B_PER_W // GATHER_CHUNK):
        sl = pl.ds(c * GATHER_CHUNK, GATHER_CHUNK)
        copies.append(pltpu.async_copy(
            user_table_hbm.at[uidx_v.at[sl]], u_rows.at[sl], sem))
        copies.append(pltpu.async_copy(
            item_table_hbm.at[iidx_v.at[sl]], i_rows.at[sl], sem))
    for cp in copies:
        cp.wait()

    lane = lax.iota(jnp.int32, L)

    def group_body(g, carry):
        rows = g * L + lane
        acc = jnp.zeros((L,), jnp.float32)
        for j in range(DIM):
            col = jnp.full((L,), j, jnp.int32)
            ug = plsc.load_gather(u_rows, [rows, col])
            ig = plsc.load_gather(i_rows, [rows, col])
            acc = acc + ug * ig
        out_v[pl.ds(g * L, L)] = acc
        return carry

    lax.fori_loop(0, B_PER_W // L, group_body, 0, unroll=False)

    pltpu.sync_copy(out_v, out_hbm.at[pl.ds(base, B_PER_W)])


@jax.jit
def _run(user_idx, item_idx, user_table, item_table):
    mesh = plsc.VectorSubcoreMesh(
        core_axis_name="c", subcore_axis_name="s",
        num_cores=NC, num_subcores=NS)
    kern = functools.partial(
        pl.kernel,
        out_type=jax.ShapeDtypeStruct((BATCH,), jnp.float32),
        mesh=mesh,
        scratch_types=[
            pltpu.VMEM((B_PER_W,), jnp.int32),
            pltpu.VMEM((B_PER_W,), jnp.int32),
            pltpu.VMEM((B_PER_W, DIM), jnp.float32),
            pltpu.VMEM((B_PER_W, DIM), jnp.float32),
            pltpu.VMEM((B_PER_W,), jnp.float32),
            pltpu.SemaphoreType.DMA,
        ],
        compiler_params=pltpu.CompilerParams(
            needs_layout_passes=False, use_tc_tiling_on_sc=False),
    )(_sc_kernel)
    return kern(user_idx, item_idx, user_table, item_table)


def kernel(user_idx, item_idx, user_table, item_table):
    return _run(user_idx.astype(jnp.int32), item_idx.astype(jnp.int32),
                user_table, item_table)


# speedup vs baseline: 3.5775x; 3.5775x over previous
import functools

import jax
import jax.numpy as jnp
from jax import lax
from jax.experimental import pallas as pl
from jax.experimental.pallas import tpu as pltpu
from jax.experimental.pallas import tpu_sc as plsc

DIM = 32
BATCH = 16384
NROWS = 1000000
NC = 2
NS = 16
NW = NC * NS
B_PER_W = BATCH // NW
G = 16
N_GROUPS = B_PER_W // G
TAIL = (NROWS // 128) * 128


def _sc_kernel(user_idx_hbm, item_idx_hbm, ut_hbm, it_hbm,
               ut_tail_hbm, it_tail_hbm,
               out_hbm, uidx_v, iidx_v, blk, rows_u, out_v, sem):
    wid = lax.axis_index("s") * NC + lax.axis_index("c")
    base = wid * B_PER_W

    pltpu.sync_copy(user_idx_hbm.at[pl.ds(base, B_PER_W)], uidx_v)
    pltpu.sync_copy(item_idx_hbm.at[pl.ds(base, B_PER_W)], iidx_v)
    pltpu.sync_copy(ut_tail_hbm, blk.at[G])
    pltpu.sync_copy(it_tail_hbm, blk.at[G + 1])

    lane = lax.iota(jnp.int32, G)

    def fetch_blocks(table_hbm, idx16, tail_slot):
        is_tail = idx16 >= TAIL
        off_vec = jnp.where(is_tail, 0, (idx16 >> 7) * 128)
        col_vec = jnp.where(is_tail, idx16 - TAIL, idx16 - off_vec)
        slot_vec = jnp.where(is_tail, tail_slot, lane)
        copies = []
        for l in range(G):
            o = pl.multiple_of(off_vec[l], 128)
            copies.append(pltpu.async_copy(
                table_hbm.at[:, pl.ds(o, 128)], blk.at[l], sem))
        for cp in copies:
            cp.wait()
        return slot_vec, col_vec

    def group_body(g, carry):
        uidx16 = uidx_v[pl.ds(g * G, G)]
        iidx16 = iidx_v[pl.ds(g * G, G)]

        uslot, ucol = fetch_blocks(ut_hbm, uidx16, G)
        for j in range(DIM):
            js = jnp.full((G,), j, jnp.int32)
            gj = plsc.load_gather(blk, [uslot, js, ucol])
            plsc.store_scatter(rows_u, [lane, js], gj)

        islot, icol = fetch_blocks(it_hbm, iidx16, G + 1)
        acc = jnp.zeros((G,), jnp.float32)
        for j in range(DIM):
            js = jnp.full((G,), j, jnp.int32)
            gi = plsc.load_gather(blk, [islot, js, icol])
            gu = plsc.load_gather(rows_u, [lane, js])
            acc = acc + gu * gi
        out_v[pl.ds(g * G, G)] = acc
        return carry

    lax.fori_loop(0, N_GROUPS, group_body, 0)

    pltpu.sync_copy(out_v, out_hbm.at[pl.ds(base, B_PER_W)])


def _pad_tail(table):
    return jnp.swapaxes(jnp.pad(table[TAIL:], ((0, 128 - (NROWS - TAIL)), (0, 0))), 0, 1)


@jax.jit
def _run(user_idx, item_idx, user_table, item_table):
    ut = jnp.swapaxes(user_table, 0, 1)
    it = jnp.swapaxes(item_table, 0, 1)
    mesh = plsc.VectorSubcoreMesh(
        core_axis_name="c", subcore_axis_name="s",
        num_cores=NC, num_subcores=NS)
    kern = functools.partial(
        pl.kernel,
        out_type=jax.ShapeDtypeStruct((BATCH,), jnp.float32),
        mesh=mesh,
        scratch_types=[
            pltpu.VMEM((B_PER_W,), jnp.int32),
            pltpu.VMEM((B_PER_W,), jnp.int32),
            pltpu.VMEM((G + 2, DIM, 128), jnp.float32),
            pltpu.VMEM((G, DIM), jnp.float32),
            pltpu.VMEM((B_PER_W,), jnp.float32),
            pltpu.SemaphoreType.DMA,
        ],
        compiler_params=pltpu.CompilerParams(
            needs_layout_passes=False, use_tc_tiling_on_sc=True),
    )(_sc_kernel)
    return kern(user_idx, item_idx, ut, it,
                _pad_tail(user_table), _pad_tail(item_table))


def kernel(user_idx, item_idx, user_table, item_table):
    return _run(user_idx.astype(jnp.int32), item_idx.astype(jnp.int32),
                user_table, item_table)
